# baseline (speedup 1.0000x reference)
import functools

import jax
import jax.numpy as jnp
from jax import lax
from jax.experimental import pallas as pl
from jax.experimental.pallas import tpu as pltpu
from jax.experimental.pallas import tpu_sc as plsc

ROWS = 16384 * 26
DIM = 64
CHUNK = 128
NBUF = 4


def _sc_geometry():
    try:
        info = plsc.get_sparse_core_info()
        return info.num_cores, info.num_subcores
    except Exception:
        return 2, 16


def _make_sc_gather(num_cores, num_subcores):
    nw = num_cores * num_subcores
    total_chunks = ROWS // CHUNK
    chunks_per_w = total_chunks // nw

    mesh = plsc.VectorSubcoreMesh(
        core_axis_name="c",
        subcore_axis_name="s",
        num_cores=num_cores,
        num_subcores=num_subcores,
    )

    @functools.partial(
        pl.kernel,
        out_type=jax.ShapeDtypeStruct((ROWS, DIM), jnp.float32),
        mesh=mesh,
        scratch_types=[
            pltpu.VMEM((chunks_per_w, CHUNK), jnp.int32),
            pltpu.VMEM((NBUF, CHUNK, DIM), jnp.float32),
            pltpu.SemaphoreType.DMA((NBUF,)),
            pltpu.SemaphoreType.DMA((NBUF,)),
        ],
        compiler_params=pltpu.CompilerParams(use_tc_tiling_on_sc=False),
    )
    def gather_kernel(idx_hbm, table_hbm, out_hbm, idx_v, rows_v, gsem, wsem):
        wid = lax.axis_index("s") * num_cores + lax.axis_index("c")
        chunk0 = wid * chunks_per_w

        pltpu.sync_copy(idx_hbm.at[pl.ds(chunk0, chunks_per_w), :], idx_v)

        def start_gather(chunk_j, buf):
            pltpu.async_copy(
                table_hbm.at[idx_v.at[chunk_j]],
                rows_v.at[buf],
                gsem.at[buf],
            )

        def wait_gather(chunk_j, buf):
            pltpu.make_async_copy(
                table_hbm.at[idx_v.at[chunk_j]],
                rows_v.at[buf],
                gsem.at[buf],
            ).wait()

        def start_write(chunk_j, buf):
            pltpu.async_copy(
                rows_v.at[buf],
                out_hbm.at[pl.ds((chunk0 + chunk_j) * CHUNK, CHUNK), :],
                wsem.at[buf],
            )

        def wait_write(chunk_j, buf):
            pltpu.make_async_copy(
                rows_v.at[buf],
                out_hbm.at[pl.ds((chunk0 + chunk_j) * CHUNK, CHUNK), :],
                wsem.at[buf],
            ).wait()

        for b in range(NBUF):
            start_gather(b, b)

        def step(i, carry):
            jbase = i * NBUF
            for b in range(NBUF):
                jj = jbase + b
                wait_gather(jj, b)
                start_write(jj, b)
                nxt = jj + NBUF

                @pl.when(nxt < chunks_per_w)
                def _():
                    wait_write(jj, b)
                    start_gather(nxt, b)

            return carry

        lax.fori_loop(0, chunks_per_w // NBUF, step, 0)

        for b in range(NBUF):
            wait_write(chunks_per_w - NBUF + b, b)

    return gather_kernel


def kernel(token_ids, weight):
    num_cores, num_subcores = _sc_geometry()
    idx = token_ids.reshape(ROWS // CHUNK, CHUNK).astype(jnp.int32)
    out = _make_sc_gather(num_cores, num_subcores)(idx, weight)
    return out.reshape(token_ids.shape + (DIM,))


# speedup vs baseline: 1.0413x; 1.0413x over previous
import functools

import jax
import jax.numpy as jnp
from jax import lax
from jax.experimental import pallas as pl
from jax.experimental.pallas import tpu as pltpu
from jax.experimental.pallas import tpu_sc as plsc

B_TOK = 16384
S_TOK = 26
ROWS = B_TOK * S_TOK
DIM = 64
CHUNK = 128
NBUF = 4


def _sc_geometry():
    try:
        info = plsc.get_sparse_core_info()
        return info.num_cores, info.num_subcores
    except Exception:
        return 2, 16


def _make_sc_gather(num_cores, num_subcores):
    nw = num_cores * num_subcores
    total_chunks = ROWS // CHUNK
    chunks_per_w = total_chunks // nw

    mesh = plsc.VectorSubcoreMesh(
        core_axis_name="c",
        subcore_axis_name="s",
        num_cores=num_cores,
        num_subcores=num_subcores,
    )

    @functools.partial(
        pl.kernel,
        out_type=jax.ShapeDtypeStruct((ROWS, DIM), jnp.float32),
        mesh=mesh,
        scratch_types=[
            pltpu.VMEM((chunks_per_w, CHUNK), jnp.int32),
            pltpu.VMEM((NBUF, CHUNK, DIM), jnp.float32),
            pltpu.SemaphoreType.DMA((NBUF,)),
            pltpu.SemaphoreType.DMA((NBUF,)),
        ],
        compiler_params=pltpu.CompilerParams(use_tc_tiling_on_sc=False),
    )
    def gather_kernel(idx_hbm, table_hbm, out_hbm, idx_v, rows_v, gsem, wsem):
        wid = lax.axis_index("s") * num_cores + lax.axis_index("c")
        chunk0 = wid * chunks_per_w

        pltpu.sync_copy(idx_hbm.at[pl.ds(chunk0, chunks_per_w), :], idx_v)

        def start_gather(chunk_j, buf):
            pltpu.async_copy(
                table_hbm.at[idx_v.at[chunk_j]],
                rows_v.at[buf],
                gsem.at[buf],
            )

        def wait_gather(chunk_j, buf):
            pltpu.make_async_copy(
                table_hbm.at[idx_v.at[chunk_j]],
                rows_v.at[buf],
                gsem.at[buf],
            ).wait()

        def start_write(chunk_j, buf):
            pltpu.async_copy(
                rows_v.at[buf],
                out_hbm.at[pl.ds((chunk0 + chunk_j) * CHUNK, CHUNK), :],
                wsem.at[buf],
            )

        def wait_write(chunk_j, buf):
            pltpu.make_async_copy(
                rows_v.at[buf],
                out_hbm.at[pl.ds((chunk0 + chunk_j) * CHUNK, CHUNK), :],
                wsem.at[buf],
            ).wait()

        for b in range(NBUF):
            start_gather(b, b)

        def step(i, carry):
            jbase = i * NBUF
            for b in range(NBUF):
                jj = jbase + b
                wait_gather(jj, b)
                start_write(jj, b)
                nxt = jj + NBUF

                @pl.when(nxt < chunks_per_w)
                def _():
                    wait_write(jj, b)
                    start_gather(nxt, b)

            return carry

        lax.fori_loop(0, chunks_per_w // NBUF, step, 0)

        for b in range(NBUF):
            wait_write(chunks_per_w - NBUF + b, b)

    return gather_kernel


def kernel(token_ids, weight):
    num_cores, num_subcores = _sc_geometry()
    idx = token_ids.T.reshape(ROWS // CHUNK, CHUNK).astype(jnp.int32)
    out_flat = _make_sc_gather(num_cores, num_subcores)(idx, weight)
    return out_flat.reshape(S_TOK, B_TOK, DIM).transpose(1, 0, 2)


# speedup vs baseline: 1.0430x; 1.0017x over previous
import functools

import jax
import jax.numpy as jnp
from jax import lax
from jax.experimental import pallas as pl
from jax.experimental.pallas import tpu as pltpu
from jax.experimental.pallas import tpu_sc as plsc

B_TOK = 16384
S_TOK = 26
ROWS = B_TOK * S_TOK
DIM = 64
CHUNK = 128
NBUF = 4
LAG = 2


def _sc_geometry():
    try:
        info = plsc.get_sparse_core_info()
        return info.num_cores, info.num_subcores
    except Exception:
        return 2, 16


def _make_sc_gather(num_cores, num_subcores):
    nw = num_cores * num_subcores
    total_chunks = ROWS // CHUNK
    chunks_per_w = total_chunks // nw
    chunks_per_s = B_TOK // CHUNK

    mesh = plsc.VectorSubcoreMesh(
        core_axis_name="c",
        subcore_axis_name="s",
        num_cores=num_cores,
        num_subcores=num_subcores,
    )

    @functools.partial(
        pl.kernel,
        out_type=jax.ShapeDtypeStruct((S_TOK, B_TOK, DIM), jnp.float32),
        mesh=mesh,
        scratch_types=[
            pltpu.VMEM((NBUF, CHUNK), jnp.int32),
            pltpu.VMEM((NBUF, CHUNK, DIM), jnp.float32),
            pltpu.SemaphoreType.DMA((NBUF,)),
            pltpu.SemaphoreType.DMA((NBUF,)),
            pltpu.SemaphoreType.DMA((NBUF,)),
        ],
        compiler_params=pltpu.CompilerParams(use_tc_tiling_on_sc=False),
    )
    def gather_kernel(idx_hbm, table_hbm, out_hbm, idx_v, rows_v,
                      isem, gsem, wsem):
        wid = lax.axis_index("s") * num_cores + lax.axis_index("c")
        chunk0 = wid * chunks_per_w

        def coords(chunk_j):
            cg = chunk0 + chunk_j
            return cg // chunks_per_s, (cg % chunks_per_s) * CHUNK

        def idx_src(chunk_j):
            s_id, b0 = coords(chunk_j)
            return idx_hbm.at[s_id, pl.ds(b0, CHUNK)]

        def start_idx(chunk_j, buf):
            pltpu.async_copy(idx_src(chunk_j), idx_v.at[buf], isem.at[buf])

        def wait_idx(chunk_j, buf):
            pltpu.make_async_copy(
                idx_src(chunk_j), idx_v.at[buf], isem.at[buf]).wait()

        def start_gather(buf):
            pltpu.async_copy(
                table_hbm.at[idx_v.at[buf]], rows_v.at[buf], gsem.at[buf])

        def wait_gather(buf):
            pltpu.make_async_copy(
                table_hbm.at[idx_v.at[buf]], rows_v.at[buf],
                gsem.at[buf]).wait()

        def out_dst(chunk_j):
            s_id, b0 = coords(chunk_j)
            return out_hbm.at[s_id, pl.ds(b0, CHUNK), :]

        def start_write(chunk_j, buf):
            pltpu.async_copy(rows_v.at[buf], out_dst(chunk_j), wsem.at[buf])

        def wait_write(chunk_j, buf):
            pltpu.make_async_copy(
                rows_v.at[buf], out_dst(chunk_j), wsem.at[buf]).wait()

        for b in range(NBUF):
            start_idx(b, b)

        def step(i, carry):
            cbase = i * NBUF
            for b in range(NBUF):
                c = cbase + b

                @pl.when(c < chunks_per_w)
                def _():
                    wait_idx(c, b)

                    @pl.when(c >= NBUF)
                    def _():
                        wait_write(c - NBUF, b)

                    start_gather(b)

                r = c - LAG
                br = (b - LAG) % NBUF

                @pl.when((r >= 0) & (r < chunks_per_w))
                def _():
                    wait_gather(br)
                    start_write(r, br)

                    @pl.when(r + NBUF < chunks_per_w)
                    def _():
                        start_idx(r + NBUF, br)

            return carry

        lax.fori_loop(0, (chunks_per_w + LAG + NBUF - 1) // NBUF, step, 0)

        for b in range(NBUF):
            wait_write(chunks_per_w - NBUF + b,
                       (chunks_per_w - NBUF + b) % NBUF)

    return gather_kernel


def kernel(token_ids, weight):
    num_cores, num_subcores = _sc_geometry()
    idx_t = token_ids.T.astype(jnp.int32)
    out3 = _make_sc_gather(num_cores, num_subcores)(idx_t, weight)
    return out3.transpose(1, 0, 2)
